# Initial kernel scaffold; baseline (speedup 1.0000x reference)
import jax
import jax.numpy as jnp
from jax.experimental import pallas as pl


def kernel(t, label, class_emb):
    raise NotImplementedError("write your pallas kernel here")


# speedup vs baseline: 1.0011x; 1.0011x over previous
import functools

import jax
import jax.numpy as jnp
from jax import lax
from jax.experimental import pallas as pl
from jax.experimental.pallas import tpu as pltpu
from jax.experimental.pallas import tpu_sc as plsc

_FDIM = 128
_BATCH = 16384
_D = _FDIM // 2

_NC = 2
_NS = 16
_NW = _NC * _NS
_BPW = _BATCH // _NW
_IDXC = 128
_NCHUNK = _BPW // _IDXC


def _sc_gather(label_r, table):
    mesh = plsc.VectorSubcoreMesh(core_axis_name="c", subcore_axis_name="s")

    @functools.partial(
        pl.kernel,
        mesh=mesh,
        out_type=jax.ShapeDtypeStruct((_BATCH, _FDIM), jnp.float32),
        scratch_types=[
            pltpu.VMEM((_NCHUNK, _IDXC), jnp.int32),
            pltpu.VMEM((_BPW, _FDIM), jnp.float32),
            pltpu.SemaphoreType.DMA,
        ],
    )
    def k(label_hbm, table_hbm, out_hbm, idx_v, rows_v, sem):
        wid = lax.axis_index("s") * _NC + lax.axis_index("c")
        base = wid * _BPW
        pltpu.sync_copy(label_hbm.at[wid], idx_v)
        copies = []
        for j in range(_NCHUNK):
            copies.append(
                pltpu.async_copy(
                    table_hbm.at[idx_v.at[j]],
                    rows_v.at[pl.ds(j * _IDXC, _IDXC)],
                    sem,
                )
            )
        for c in copies:
            c.wait()
        pltpu.sync_copy(rows_v, out_hbm.at[pl.ds(base, _BPW)])

    return k(label_r, table)


def _tc_body(t_ref, denom_ref, g_ref, o_ref):
    targ = t_ref[...] / denom_ref[...]
    emb = jnp.concatenate((jnp.sin(targ), jnp.cos(targ)), axis=1)
    o_ref[...] = emb + g_ref[...]


def _tc_sin_add(t2, denom2, g):
    bb = 2048
    return pl.pallas_call(
        _tc_body,
        out_shape=jax.ShapeDtypeStruct((_BATCH, _FDIM), jnp.float32),
        grid=(_BATCH // bb,),
        in_specs=[
            pl.BlockSpec((bb, 1), lambda i: (i, 0)),
            pl.BlockSpec((1, _D), lambda i: (0, 0)),
            pl.BlockSpec((bb, _FDIM), lambda i: (i, 0)),
        ],
        out_specs=pl.BlockSpec((bb, _FDIM), lambda i: (i, 0)),
    )(t2, denom2, g)


def kernel(t, label, class_emb):
    label_r = label.astype(jnp.int32).reshape(_NW, _NCHUNK, _IDXC)
    gathered = _sc_gather(label_r, class_emb)
    denom = 10000.0 ** (jnp.arange(_D, dtype=jnp.float32) / (_D - 1))
    return _tc_sin_add(t.reshape(_BATCH, 1), denom.reshape(1, _D), gathered)
